# baseline (speedup 1.0000x reference)
import functools

import jax
import jax.numpy as jnp
from jax import lax
from jax.experimental import pallas as pl
from jax.experimental.pallas import tpu as pltpu
from jax.experimental.pallas import tpu_sc as plsc

N_E = 1024
E_DIM = 256
BETA = 0.25
B = 16
HW = 1024
T = 512
NT = HW // T

NC = 2
NS = 16
NW = NC * NS
TPW = B * HW // NW
CHUNK = 128
NCHUNK = TPW // CHUNK


def _vq_dist_block(z_ref, cb_ref, idx_ref):
    zb = z_ref[0]
    zt = zb.T
    cb = cb_ref[...]

    zn = jnp.sum(zt * zt, axis=1, keepdims=True)
    cbn = jnp.sum(cb * cb, axis=1)
    mm = lax.dot_general(zt, cb, (((1,), (1,)), ((), ())),
                         preferred_element_type=jnp.float32)
    d = zn + cbn[None, :] - 2.0 * mm

    minval = jnp.min(d, axis=1, keepdims=True)
    iota_f = lax.broadcasted_iota(jnp.int32, (T, N_E), 1).astype(jnp.float32)
    idxf = jnp.min(jnp.where(d == minval, iota_f, float(N_E)), axis=1,
                   keepdims=True)
    idx_ref[0, 0] = idxf[:, 0].astype(jnp.int32)


def _tc_distances(z3, codebook):
    grid = (B * NT,)
    return pl.pallas_call(
        _vq_dist_block,
        grid=grid,
        in_specs=[
            pl.BlockSpec((1, E_DIM, T), lambda i: (i // NT, 0, i % NT)),
            pl.BlockSpec((N_E, E_DIM), lambda i: (0, 0)),
        ],
        out_specs=pl.BlockSpec((1, 1, T), lambda i: (i, 0, 0)),
        out_shape=jax.ShapeDtypeStruct((B * NT, 1, T), jnp.int32),
    )(z3, codebook)


def _sc_gather_body(cb_hbm, idx_hbm, out_hbm, idx_v, buf0, buf1, sem0, sem1):
    wid = lax.axis_index("s") * NC + lax.axis_index("c")
    tok0 = wid * TPW
    pltpu.sync_copy(idx_hbm.at[pl.ds(tok0, TPW)], idx_v)

    bufs = (buf0, buf1)
    sems = (sem0, sem1)
    for k in range(NCHUNK):
        buf, sem = bufs[k % 2], sems[k % 2]
        g = pltpu.make_async_copy(
            cb_hbm.at[idx_v.at[pl.ds(k * CHUNK, CHUNK)]], buf, sem)
        g.start()
        g.wait()
        pltpu.sync_copy(buf, out_hbm.at[pl.ds(tok0 + k * CHUNK, CHUNK)])


def _sc_gather(codebook, idx_flat):
    mesh = plsc.VectorSubcoreMesh(core_axis_name="c", subcore_axis_name="s")
    f = functools.partial(
        pl.kernel,
        mesh=mesh,
        out_type=jax.ShapeDtypeStruct((B * HW, E_DIM), jnp.float32),
        scratch_types=[
            pltpu.VMEM((TPW,), jnp.int32),
            pltpu.VMEM((CHUNK, E_DIM), jnp.float32),
            pltpu.VMEM((CHUNK, E_DIM), jnp.float32),
            pltpu.SemaphoreType.DMA,
            pltpu.SemaphoreType.DMA,
        ],
    )(_sc_gather_body)
    return f(codebook, idx_flat)


def _transpose_block(zq_ref, z_ref, out_ref, part_ref):
    zq = zq_ref[0].T
    out_ref[0] = zq
    diff = zq - z_ref[0]
    part_ref[0, 0] = jnp.broadcast_to(jnp.sum(diff * diff), (128,))


def _tc_transpose(zq_rows, z3):
    return pl.pallas_call(
        _transpose_block,
        grid=(B,),
        in_specs=[
            pl.BlockSpec((1, HW, E_DIM), lambda i: (i, 0, 0)),
            pl.BlockSpec((1, E_DIM, HW), lambda i: (i, 0, 0)),
        ],
        out_specs=[
            pl.BlockSpec((1, E_DIM, HW), lambda i: (i, 0, 0)),
            pl.BlockSpec((1, 1, 128), lambda i: (i, 0, 0)),
        ],
        out_shape=[
            jax.ShapeDtypeStruct((B, E_DIM, HW), jnp.float32),
            jax.ShapeDtypeStruct((B, 1, 128), jnp.float32),
        ],
    )(zq_rows, z3)


def kernel(z, codebook):
    z3 = z.reshape(B, E_DIM, HW)
    idx2 = _tc_distances(z3, codebook)
    zq_rows = _sc_gather(codebook, idx2.reshape(B * HW))
    zq3, parts = _tc_transpose(zq_rows.reshape(B, HW, E_DIM), z3)
    z_q_out = zq3.reshape(z.shape)
    loss = (1.0 + BETA) * jnp.sum(parts[:, 0, 0]) / (B * HW * E_DIM)
    indices_out = idx2.reshape(B, 1, 32, 32)
    return (z_q_out, loss, indices_out)


# speedup vs baseline: 1.0080x; 1.0080x over previous
import functools

import jax
import jax.numpy as jnp
from jax import lax
from jax.experimental import pallas as pl
from jax.experimental.pallas import tpu as pltpu
from jax.experimental.pallas import tpu_sc as plsc

N_E = 1024
E_DIM = 256
BETA = 0.25
B = 16
HW = 1024
T = 512
NT = HW // T

NC = 2
NS = 16
NW = NC * NS
TPW = B * HW // NW
CHUNK = 128
NCHUNK = TPW // CHUNK


def _vq_dist_block(z_ref, cb_ref, idx_ref):
    zb = z_ref[0]
    zt = zb.T
    cb = cb_ref[...]

    zn = jnp.sum(zt * zt, axis=1, keepdims=True)
    cbn = jnp.sum(cb * cb, axis=1)
    mm = lax.dot_general(zt, cb, (((1,), (1,)), ((), ())),
                         preferred_element_type=jnp.float32)
    d = zn + cbn[None, :] - 2.0 * mm

    minval = jnp.min(d, axis=1, keepdims=True)
    iota_f = lax.broadcasted_iota(jnp.int32, (T, N_E), 1).astype(jnp.float32)
    idxf = jnp.min(jnp.where(d == minval, iota_f, float(N_E)), axis=1,
                   keepdims=True)
    idx_ref[0, 0] = idxf[:, 0].astype(jnp.int32)


def _tc_distances(z3, codebook):
    grid = (B * NT,)
    return pl.pallas_call(
        _vq_dist_block,
        grid=grid,
        in_specs=[
            pl.BlockSpec((1, E_DIM, T), lambda i: (i // NT, 0, i % NT)),
            pl.BlockSpec((N_E, E_DIM), lambda i: (0, 0)),
        ],
        out_specs=pl.BlockSpec((1, 1, T), lambda i: (i, 0, 0)),
        out_shape=jax.ShapeDtypeStruct((B * NT, 1, T), jnp.int32),
    )(z3, codebook)


def _sc_gather_body(cb_hbm, idx_hbm, out_hbm, idx_v, buf0, buf1, sem0, sem1,
                    sem2, sem3):
    wid = lax.axis_index("s") * NC + lax.axis_index("c")
    tok0 = wid * TPW
    pltpu.sync_copy(idx_hbm.at[pl.ds(tok0, TPW)], idx_v)

    bufs = (buf0, buf1)
    gsems = (sem0, sem1)
    ssems = (sem2, sem3)
    gathers = [None, None]
    stores = [None, None]
    for k in range(NCHUNK):
        p = k % 2
        if stores[p] is not None:
            stores[p].wait()
        g = pltpu.make_async_copy(
            cb_hbm.at[idx_v.at[pl.ds(k * CHUNK, CHUNK)]], bufs[p], gsems[p])
        g.start()
        gathers[p] = g
        q = 1 - p
        if gathers[q] is not None:
            gathers[q].wait()
            s = pltpu.make_async_copy(
                bufs[q], out_hbm.at[pl.ds(tok0 + (k - 1) * CHUNK, CHUNK)],
                ssems[q])
            s.start()
            stores[q] = s
            gathers[q] = None
    p = (NCHUNK - 1) % 2
    gathers[p].wait()
    s = pltpu.make_async_copy(
        bufs[p], out_hbm.at[pl.ds(tok0 + (NCHUNK - 1) * CHUNK, CHUNK)],
        ssems[p])
    s.start()
    stores[p] = s
    for st in stores:
        if st is not None:
            st.wait()


def _sc_gather(codebook, idx_flat):
    mesh = plsc.VectorSubcoreMesh(core_axis_name="c", subcore_axis_name="s")
    f = functools.partial(
        pl.kernel,
        mesh=mesh,
        out_type=jax.ShapeDtypeStruct((B * HW, E_DIM), jnp.float32),
        scratch_types=[
            pltpu.VMEM((TPW,), jnp.int32),
            pltpu.VMEM((CHUNK, E_DIM), jnp.float32),
            pltpu.VMEM((CHUNK, E_DIM), jnp.float32),
            pltpu.SemaphoreType.DMA,
            pltpu.SemaphoreType.DMA,
            pltpu.SemaphoreType.DMA,
            pltpu.SemaphoreType.DMA,
        ],
    )(_sc_gather_body)
    return f(codebook, idx_flat)


def _transpose_block(zq_ref, z_ref, out_ref, part_ref):
    zq = zq_ref[0].T
    out_ref[0] = zq
    diff = zq - z_ref[0]
    part_ref[0, 0] = jnp.broadcast_to(jnp.sum(diff * diff), (128,))


def _tc_transpose(zq_rows, z3):
    return pl.pallas_call(
        _transpose_block,
        grid=(B,),
        in_specs=[
            pl.BlockSpec((1, HW, E_DIM), lambda i: (i, 0, 0)),
            pl.BlockSpec((1, E_DIM, HW), lambda i: (i, 0, 0)),
        ],
        out_specs=[
            pl.BlockSpec((1, E_DIM, HW), lambda i: (i, 0, 0)),
            pl.BlockSpec((1, 1, 128), lambda i: (i, 0, 0)),
        ],
        out_shape=[
            jax.ShapeDtypeStruct((B, E_DIM, HW), jnp.float32),
            jax.ShapeDtypeStruct((B, 1, 128), jnp.float32),
        ],
    )(zq_rows, z3)


def kernel(z, codebook):
    z3 = z.reshape(B, E_DIM, HW)
    idx2 = _tc_distances(z3, codebook)
    zq_rows = _sc_gather(codebook, idx2.reshape(B * HW))
    zq3, parts = _tc_transpose(zq_rows.reshape(B, HW, E_DIM), z3)
    z_q_out = zq3.reshape(z.shape)
    loss = (1.0 + BETA) * jnp.sum(parts[:, 0, 0]) / (B * HW * E_DIM)
    indices_out = idx2.reshape(B, 1, 32, 32)
    return (z_q_out, loss, indices_out)


# speedup vs baseline: 1.7106x; 1.6969x over previous
import functools

import jax
import jax.numpy as jnp
from jax import lax
from jax.experimental import pallas as pl

N_E = 1024
E_DIM = 256
BETA = 0.25
B = 16
HW = 1024
T = 512
NT = HW // T


def _vq_block(z_ref, cb_ref, zq_ref, idx_ref, part_ref):
    zb = z_ref[0]
    zt = zb.T
    cb = cb_ref[...]

    zn = jnp.sum(zt * zt, axis=1, keepdims=True)
    cbn = jnp.sum(cb * cb, axis=1)
    mm = lax.dot_general(zt, cb, (((1,), (1,)), ((), ())),
                         preferred_element_type=jnp.float32)
    d = zn + cbn[None, :] - 2.0 * mm

    minval = jnp.min(d, axis=1, keepdims=True)
    iota_f = lax.broadcasted_iota(jnp.int32, (T, N_E), 1).astype(jnp.float32)
    idxf = jnp.min(jnp.where(d == minval, iota_f, float(N_E)), axis=1,
                   keepdims=True)
    idx_ref[0, 0] = idxf[:, 0].astype(jnp.int32)

    onehot = jnp.where(iota_f == idxf, 1.0, 0.0)
    zq = lax.dot_general(cb, onehot, (((0,), (1,)), ((), ())),
                         preferred_element_type=jnp.float32)
    zq_ref[0] = zq

    diff = zq - zb
    part = jnp.sum(diff * diff)
    part_ref[0, 0] = jnp.broadcast_to(part, (128,))


@functools.partial(jax.jit, static_argnames=())
def kernel(z, codebook):
    z3 = z.reshape(B, E_DIM, HW)
    grid = (B * NT,)

    zq3, idx2, parts = pl.pallas_call(
        _vq_block,
        grid=grid,
        in_specs=[
            pl.BlockSpec((1, E_DIM, T), lambda i: (i // NT, 0, i % NT)),
            pl.BlockSpec((N_E, E_DIM), lambda i: (0, 0)),
        ],
        out_specs=[
            pl.BlockSpec((1, E_DIM, T), lambda i: (i // NT, 0, i % NT)),
            pl.BlockSpec((1, 1, T), lambda i: (i, 0, 0)),
            pl.BlockSpec((1, 1, 128), lambda i: (i, 0, 0)),
        ],
        out_shape=[
            jax.ShapeDtypeStruct((B, E_DIM, HW), jnp.float32),
            jax.ShapeDtypeStruct((B * NT, 1, T), jnp.int32),
            jax.ShapeDtypeStruct((B * NT, 1, 128), jnp.float32),
        ],
    )(z3, codebook)

    z_q_out = zq3.reshape(z.shape)
    loss = (1.0 + BETA) * jnp.sum(parts[:, 0, 0]) / (B * HW * E_DIM)
    indices_out = idx2.reshape(B, 1, 32, 32)
    return (z_q_out, loss, indices_out)


# speedup vs baseline: 1.8272x; 1.0682x over previous
import functools

import jax
import jax.numpy as jnp
from jax import lax
from jax.experimental import pallas as pl

N_E = 1024
E_DIM = 256
BETA = 0.25
B = 16
HW = 1024
T = 1024
NT = HW // T


def _vq_block(z_ref, cb_ref, zq_ref, idx_ref, part_ref):
    zb = z_ref[0]
    zt = zb.T
    cb = cb_ref[...]

    zn = jnp.sum(zt * zt, axis=1, keepdims=True)
    cbn = jnp.sum(cb * cb, axis=1)
    mm = lax.dot_general(zt, cb, (((1,), (1,)), ((), ())),
                         preferred_element_type=jnp.float32)
    d = zn + cbn[None, :] - 2.0 * mm

    minval = jnp.min(d, axis=1, keepdims=True)
    iota_f = lax.broadcasted_iota(jnp.int32, (T, N_E), 1).astype(jnp.float32)
    idxf = jnp.min(jnp.where(d == minval, iota_f, float(N_E)), axis=1,
                   keepdims=True)
    idx_ref[0, 0] = idxf[:, 0].astype(jnp.int32)

    onehot = jnp.where(iota_f == idxf, 1.0, 0.0)
    zq = lax.dot_general(cb, onehot, (((0,), (1,)), ((), ())),
                         preferred_element_type=jnp.float32)
    zq_ref[0] = zq

    diff = zq - zb
    part = jnp.sum(diff * diff)
    part_ref[0, 0] = jnp.broadcast_to(part, (128,))


@functools.partial(jax.jit, static_argnames=())
def kernel(z, codebook):
    z3 = z.reshape(B, E_DIM, HW)
    grid = (B * NT,)

    zq3, idx2, parts = pl.pallas_call(
        _vq_block,
        grid=grid,
        in_specs=[
            pl.BlockSpec((1, E_DIM, T), lambda i: (i // NT, 0, i % NT)),
            pl.BlockSpec((N_E, E_DIM), lambda i: (0, 0)),
        ],
        out_specs=[
            pl.BlockSpec((1, E_DIM, T), lambda i: (i // NT, 0, i % NT)),
            pl.BlockSpec((1, 1, T), lambda i: (i, 0, 0)),
            pl.BlockSpec((1, 1, 128), lambda i: (i, 0, 0)),
        ],
        out_shape=[
            jax.ShapeDtypeStruct((B, E_DIM, HW), jnp.float32),
            jax.ShapeDtypeStruct((B * NT, 1, T), jnp.int32),
            jax.ShapeDtypeStruct((B * NT, 1, 128), jnp.float32),
        ],
    )(z3, codebook)

    z_q_out = zq3.reshape(z.shape)
    loss = (1.0 + BETA) * jnp.sum(parts[:, 0, 0]) / (B * HW * E_DIM)
    indices_out = idx2.reshape(B, 1, 32, 32)
    return (z_q_out, loss, indices_out)


# speedup vs baseline: 2.0749x; 1.1355x over previous
import functools

import jax
import jax.numpy as jnp
from jax import lax
from jax.experimental import pallas as pl

N_E = 1024
E_DIM = 256
BETA = 0.25
B = 16
HW = 1024
T = 1024
NT = HW // T


def _vq_block(z_ref, cb_ref, zq_ref, idx_ref, part_ref):
    zb = z_ref[0]
    cb = cb_ref[...]

    zn = jnp.sum(zb * zb, axis=0, keepdims=True)
    cbn = jnp.sum(cb * cb, axis=1)
    mmT = lax.dot_general(cb, zb, (((1,), (0,)), ((), ())),
                          preferred_element_type=jnp.float32)
    d = zn + cbn[:, None] - 2.0 * mmT

    minval = jnp.min(d, axis=0, keepdims=True)
    iota_f = lax.broadcasted_iota(jnp.int32, (N_E, T), 0).astype(jnp.float32)
    idxf = jnp.min(jnp.where(d == minval, iota_f, float(N_E)), axis=0,
                   keepdims=True)
    idx_ref[0, 0] = idxf[0, :].astype(jnp.int32)

    onehot = jnp.where(iota_f == idxf, 1.0, 0.0)
    zq = lax.dot_general(cb, onehot, (((0,), (0,)), ((), ())),
                         preferred_element_type=jnp.float32)
    zq_ref[0] = zq

    diff = zq - zb
    part = jnp.sum(diff * diff)
    part_ref[0, 0] = jnp.broadcast_to(part, (128,))


@functools.partial(jax.jit, static_argnames=())
def kernel(z, codebook):
    z3 = z.reshape(B, E_DIM, HW)
    grid = (B * NT,)

    zq3, idx2, parts = pl.pallas_call(
        _vq_block,
        grid=grid,
        in_specs=[
            pl.BlockSpec((1, E_DIM, T), lambda i: (i // NT, 0, i % NT)),
            pl.BlockSpec((N_E, E_DIM), lambda i: (0, 0)),
        ],
        out_specs=[
            pl.BlockSpec((1, E_DIM, T), lambda i: (i // NT, 0, i % NT)),
            pl.BlockSpec((1, 1, T), lambda i: (i, 0, 0)),
            pl.BlockSpec((1, 1, 128), lambda i: (i, 0, 0)),
        ],
        out_shape=[
            jax.ShapeDtypeStruct((B, E_DIM, HW), jnp.float32),
            jax.ShapeDtypeStruct((B * NT, 1, T), jnp.int32),
            jax.ShapeDtypeStruct((B * NT, 1, 128), jnp.float32),
        ],
    )(z3, codebook)

    z_q_out = zq3.reshape(z.shape)
    loss = (1.0 + BETA) * jnp.sum(parts[:, 0, 0]) / (B * HW * E_DIM)
    indices_out = idx2.reshape(B, 1, 32, 32)
    return (z_q_out, loss, indices_out)
